# baseline (speedup 1.0000x reference)
import functools

import jax
import jax.numpy as jnp
from jax import lax
from jax.experimental import pallas as pl
from jax.experimental.pallas import tpu as pltpu
from jax.experimental.pallas import tpu_sc as plsc

_BT = 256
_A = 100000
_R = 8
_CH = 512
_NCH = _A // _CH
_TAIL = _A - _NCH * _CH
_TAIL0 = _NCH * _CH

_KS0 = 0
_KS1 = 42
_KS2 = 0 ^ 42 ^ 0x1BD11BDA
_KS = (_KS0, _KS1, _KS2)
_ROTS = ((13, 15, 26, 6), (17, 29, 16, 24))
_TINY = float(jnp.finfo(jnp.float32).tiny)
_NEG_INF = float("-inf")
_IMAX = jnp.iinfo(jnp.int32).max


def _rotl(x, r):
    return lax.shift_left(x, r) | lax.shift_right_logical(x, 32 - r)


def _gumbel_bits(i):
    x1 = i + _KS1
    x0 = x1
    first = True
    for r in range(5):
        for rot in _ROTS[r % 2]:
            if first:
                first = False
            else:
                x0 = x0 + x1
            x1 = _rotl(x1, rot)
            x1 = x1 ^ x0
        x0 = x0 + _KS[(r + 1) % 3]
        x1 = x1 + (_KS[(r + 2) % 3] + r + 1)
    bits = x0 ^ x1
    fbits = lax.shift_right_logical(bits, 9) | 0x3F800000
    f = lax.bitcast_convert_type(fbits, jnp.float32) - 1.0
    u = jnp.maximum(_TINY, f * (1.0 - _TINY) + _TINY)
    return -jnp.log(-jnp.log(u))


def _sample_body(x_ref, lp_ref, idx_ref, alp_ref):
    b = pl.program_id(0)
    rowbase = (lax.broadcasted_iota(jnp.int32, (_R, 1), 0) + b * _R) * _A
    colc = lax.broadcasted_iota(jnp.int32, (_R, _CH), 1)
    colt = lax.broadcasted_iota(jnp.int32, (_R, _TAIL), 1)

    def max_body(c, mp):
        xs = x_ref[:, pl.ds(c * _CH, _CH)]
        return jnp.maximum(mp, xs)

    mp = jax.lax.fori_loop(
        0, _NCH, max_body, jnp.full((_R, _CH), _NEG_INF, jnp.float32))
    m = jnp.max(mp, axis=1, keepdims=True)
    xt = x_ref[:, pl.ds(_TAIL0, _TAIL)]
    m = jnp.maximum(m, jnp.max(xt, axis=1, keepdims=True))

    def samp_body(c, carry):
        sp, best, bestcol = carry
        xs = x_ref[:, pl.ds(c * _CH, _CH)]
        col = colc + c * _CH
        g = _gumbel_bits(rowbase + col)
        y = xs + g
        upd = y > best
        best = jnp.where(upd, y, best)
        bestcol = jnp.where(upd, col, bestcol)
        sp = sp + jnp.exp(xs - m)
        return sp, best, bestcol

    sp, best, bestcol = jax.lax.fori_loop(
        0, _NCH, samp_body,
        (jnp.zeros((_R, _CH), jnp.float32),
         jnp.full((_R, _CH), _NEG_INF, jnp.float32),
         jnp.zeros((_R, _CH), jnp.int32)))

    s = jnp.sum(sp, axis=1, keepdims=True)
    s = s + jnp.sum(jnp.exp(xt - m), axis=1, keepdims=True)
    lse = m + jnp.log(s)

    maxv = jnp.max(best, axis=1, keepdims=True)
    idx_m = jnp.min(jnp.where(best == maxv, bestcol, _IMAX),
                    axis=1, keepdims=True)
    yt = xt + _gumbel_bits(rowbase + _TAIL0 + colt)
    maxv_t = jnp.max(yt, axis=1, keepdims=True)
    idx_t = jnp.min(jnp.where(yt == maxv_t, colt + _TAIL0, _IMAX),
                    axis=1, keepdims=True)
    tw = maxv_t > maxv
    idx = jnp.where(tw, idx_t, idx_m)
    idx_ref[...] = idx

    def write_body(c, ap):
        xs = x_ref[:, pl.ds(c * _CH, _CH)]
        col = colc + c * _CH
        lp = xs - lse
        lp_ref[:, pl.ds(c * _CH, _CH)] = lp
        return jnp.maximum(ap, jnp.where(col == idx, lp, _NEG_INF))

    ap = jax.lax.fori_loop(
        0, _NCH, write_body, jnp.full((_R, _CH), _NEG_INF, jnp.float32))
    alp = jnp.max(ap, axis=1, keepdims=True)
    lpt = xt - lse
    lp_ref[:, pl.ds(_TAIL0, _TAIL)] = lpt
    alp_t = jnp.max(jnp.where(colt + _TAIL0 == idx, lpt, _NEG_INF),
                    axis=1, keepdims=True)
    alp_ref[...] = jnp.maximum(alp, alp_t)


def _sample(logits, interpret=False):
    return pl.pallas_call(
        _sample_body,
        grid=(_BT // _R,),
        in_specs=[pl.BlockSpec((_R, _A), lambda b: (b, 0))],
        out_specs=[
            pl.BlockSpec((_R, _A), lambda b: (b, 0)),
            pl.BlockSpec((_R, 1), lambda b: (b, 0)),
            pl.BlockSpec((_R, 1), lambda b: (b, 0)),
        ],
        out_shape=[
            jax.ShapeDtypeStruct((_BT, _A), jnp.float32),
            jax.ShapeDtypeStruct((_BT, 1), jnp.int32),
            jax.ShapeDtypeStruct((_BT, 1), jnp.float32),
        ],
        compiler_params=pltpu.CompilerParams(
            dimension_semantics=("parallel",),
        ),
        interpret=interpret,
    )(logits)


def kernel(logits, value, action_index_tensor):
    lp, idx, alp = _sample(logits)
    idx = idx.reshape(-1)
    actions = jnp.take(action_index_tensor, idx, axis=0)
    return actions, alp.reshape(-1), value.reshape(-1), lp


# speedup vs baseline: 1.3187x; 1.3187x over previous
import functools

import jax
import jax.numpy as jnp
from jax import lax
from jax.experimental import pallas as pl
from jax.experimental.pallas import tpu as pltpu
from jax.experimental.pallas import tpu_sc as plsc

_BT = 256
_A = 100000
_R = 8
_CH = 512
_U = 2
_NCH = _A // _CH
_TAIL = _A - _NCH * _CH
_TAIL0 = _NCH * _CH

_KS0 = 0
_KS1 = 42
_KS2 = 0 ^ 42 ^ 0x1BD11BDA
_KS = (_KS0, _KS1, _KS2)
_ROTS = ((13, 15, 26, 6), (17, 29, 16, 24))
_TINY = float(jnp.finfo(jnp.float32).tiny)
_NEG_INF = float("-inf")
_IMAX = jnp.iinfo(jnp.int32).max


def _rotl(x, r):
    return lax.shift_left(x, r) | lax.shift_right_logical(x, 32 - r)


def _gumbel_bits(i):
    x1 = i + _KS1
    x0 = x1
    first = True
    for r in range(5):
        for rot in _ROTS[r % 2]:
            if first:
                first = False
            else:
                x0 = x0 + x1
            x1 = _rotl(x1, rot)
            x1 = x1 ^ x0
        x0 = x0 + _KS[(r + 1) % 3]
        x1 = x1 + (_KS[(r + 2) % 3] + r + 1)
    bits = x0 ^ x1
    fbits = lax.shift_right_logical(bits, 9) | 0x3F800000
    f = lax.bitcast_convert_type(fbits, jnp.float32) - 1.0
    u = jnp.maximum(_TINY, f * (1.0 - _TINY) + _TINY)
    return -jnp.log(-jnp.log(u))


def _sample_body(x_ref, lp_ref, idx_ref, alp_ref):
    b = pl.program_id(0)
    rowbase = (lax.broadcasted_iota(jnp.int32, (_R, 1), 0) + b * _R) * _A
    colc = lax.broadcasted_iota(jnp.int32, (_R, _CH), 1)
    colt = lax.broadcasted_iota(jnp.int32, (_R, _TAIL), 1)

    def max_body(c, mp):
        xs = x_ref[:, pl.ds(c * _CH, _CH)]
        return jnp.maximum(mp, xs)

    mp = jax.lax.fori_loop(
        0, _NCH, max_body, jnp.full((_R, _CH), _NEG_INF, jnp.float32))
    m = jnp.max(mp, axis=1, keepdims=True)
    xt = x_ref[:, pl.ds(_TAIL0, _TAIL)]
    m = jnp.maximum(m, jnp.max(xt, axis=1, keepdims=True))

    def chunk_upd(cc, carry):
        sp, best, bestcol = carry
        xs = x_ref[:, pl.ds(cc * _CH, _CH)]
        col = colc + cc * _CH
        y = xs + _gumbel_bits(rowbase + col)
        upd = y > best
        best = jnp.where(upd, y, best)
        bestcol = jnp.where(upd, col, bestcol)
        sp = sp + jnp.exp(xs - m)
        return sp, best, bestcol

    def samp_body(c, carry):
        for k in range(_U):
            carry = chunk_upd(c * _U + k, carry)
        return carry

    carry0 = (jnp.zeros((_R, _CH), jnp.float32),
              jnp.full((_R, _CH), _NEG_INF, jnp.float32),
              jnp.zeros((_R, _CH), jnp.int32))
    carry = jax.lax.fori_loop(0, _NCH // _U, samp_body, carry0)
    for cc in range(_NCH - _NCH % _U, _NCH):
        carry = chunk_upd(cc, carry)
    sp, best, bestcol = carry

    s = jnp.sum(sp, axis=1, keepdims=True)
    s = s + jnp.sum(jnp.exp(xt - m), axis=1, keepdims=True)
    lse = m + jnp.log(s)

    maxv = jnp.max(best, axis=1, keepdims=True)
    idx_m = jnp.min(jnp.where(best == maxv, bestcol, _IMAX),
                    axis=1, keepdims=True)
    yt = xt + _gumbel_bits(rowbase + _TAIL0 + colt)
    maxv_t = jnp.max(yt, axis=1, keepdims=True)
    idx_t = jnp.min(jnp.where(yt == maxv_t, colt + _TAIL0, _IMAX),
                    axis=1, keepdims=True)
    tw = maxv_t > maxv
    idx = jnp.where(tw, idx_t, idx_m)
    idx_ref[...] = idx

    def write_body(c, ap):
        xs = x_ref[:, pl.ds(c * _CH, _CH)]
        col = colc + c * _CH
        lp = xs - lse
        lp_ref[:, pl.ds(c * _CH, _CH)] = lp
        return jnp.maximum(ap, jnp.where(col == idx, lp, _NEG_INF))

    ap = jax.lax.fori_loop(
        0, _NCH, write_body, jnp.full((_R, _CH), _NEG_INF, jnp.float32))
    alp = jnp.max(ap, axis=1, keepdims=True)
    lpt = xt - lse
    lp_ref[:, pl.ds(_TAIL0, _TAIL)] = lpt
    alp_t = jnp.max(jnp.where(colt + _TAIL0 == idx, lpt, _NEG_INF),
                    axis=1, keepdims=True)
    alp_ref[...] = jnp.maximum(alp, alp_t)


def _sample(logits, interpret=False):
    return pl.pallas_call(
        _sample_body,
        grid=(_BT // _R,),
        in_specs=[pl.BlockSpec((_R, _A), lambda b: (b, 0))],
        out_specs=[
            pl.BlockSpec((_R, _A), lambda b: (b, 0)),
            pl.BlockSpec((_R, 1), lambda b: (b, 0)),
            pl.BlockSpec((_R, 1), lambda b: (b, 0)),
        ],
        out_shape=[
            jax.ShapeDtypeStruct((_BT, _A), jnp.float32),
            jax.ShapeDtypeStruct((_BT, 1), jnp.int32),
            jax.ShapeDtypeStruct((_BT, 1), jnp.float32),
        ],
        compiler_params=pltpu.CompilerParams(
            dimension_semantics=("parallel",),
        ),
        interpret=interpret,
    )(logits)


def kernel(logits, value, action_index_tensor):
    lp, idx, alp = _sample(logits)
    idx = idx.reshape(-1)
    actions = jnp.take(action_index_tensor, idx, axis=0)
    return actions, alp.reshape(-1), value.reshape(-1), lp


# speedup vs baseline: 1.3770x; 1.0442x over previous
import functools

import jax
import jax.numpy as jnp
from jax import lax
from jax.experimental import pallas as pl
from jax.experimental.pallas import tpu as pltpu
from jax.experimental.pallas import tpu_sc as plsc

_BT = 256
_A = 100000
_R = 8
_CH = 512
_U = 4
_NCH = _A // _CH
_TAIL = _A - _NCH * _CH
_TAIL0 = _NCH * _CH

_KS0 = 0
_KS1 = 42
_KS2 = 0 ^ 42 ^ 0x1BD11BDA
_KS = (_KS0, _KS1, _KS2)
_ROTS = ((13, 15, 26, 6), (17, 29, 16, 24))
_TINY = float(jnp.finfo(jnp.float32).tiny)
_NEG_INF = float("-inf")
_IMAX = jnp.iinfo(jnp.int32).max


def _rotl(x, r):
    return lax.shift_left(x, r) | lax.shift_right_logical(x, 32 - r)


def _gumbel_bits(i):
    x1 = i + _KS1
    x0 = x1
    first = True
    for r in range(5):
        for rot in _ROTS[r % 2]:
            if first:
                first = False
            else:
                x0 = x0 + x1
            x1 = _rotl(x1, rot)
            x1 = x1 ^ x0
        x0 = x0 + _KS[(r + 1) % 3]
        x1 = x1 + (_KS[(r + 2) % 3] + r + 1)
    bits = x0 ^ x1
    fbits = lax.shift_right_logical(bits, 9) | 0x3F800000
    f = lax.bitcast_convert_type(fbits, jnp.float32) - 1.0
    u = jnp.maximum(_TINY, f * (1.0 - _TINY) + _TINY)
    return -jnp.log(-jnp.log(u))


def _sample_body(x_ref, lp_ref, idx_ref, alp_ref):
    b = pl.program_id(0)
    rowbase = (lax.broadcasted_iota(jnp.int32, (_R, 1), 0) + b * _R) * _A
    colc = lax.broadcasted_iota(jnp.int32, (_R, _CH), 1)
    colt = lax.broadcasted_iota(jnp.int32, (_R, _TAIL), 1)

    def max_body(c, mp):
        xs = x_ref[:, pl.ds(c * _CH, _CH)]
        return jnp.maximum(mp, xs)

    mp = jax.lax.fori_loop(
        0, _NCH, max_body, jnp.full((_R, _CH), _NEG_INF, jnp.float32))
    m = jnp.max(mp, axis=1, keepdims=True)
    xt = x_ref[:, pl.ds(_TAIL0, _TAIL)]
    m = jnp.maximum(m, jnp.max(xt, axis=1, keepdims=True))

    def chunk_upd(cc, carry):
        sp, best, bestcol = carry
        xs = x_ref[:, pl.ds(cc * _CH, _CH)]
        col = colc + cc * _CH
        y = xs + _gumbel_bits(rowbase + col)
        upd = y > best
        best = jnp.where(upd, y, best)
        bestcol = jnp.where(upd, col, bestcol)
        sp = sp + jnp.exp(xs - m)
        return sp, best, bestcol

    def samp_body(c, carry):
        for k in range(_U):
            carry = chunk_upd(c * _U + k, carry)
        return carry

    carry0 = (jnp.zeros((_R, _CH), jnp.float32),
              jnp.full((_R, _CH), _NEG_INF, jnp.float32),
              jnp.zeros((_R, _CH), jnp.int32))
    carry = jax.lax.fori_loop(0, _NCH // _U, samp_body, carry0)
    for cc in range(_NCH - _NCH % _U, _NCH):
        carry = chunk_upd(cc, carry)
    sp, best, bestcol = carry

    s = jnp.sum(sp, axis=1, keepdims=True)
    s = s + jnp.sum(jnp.exp(xt - m), axis=1, keepdims=True)
    lse = m + jnp.log(s)

    maxv = jnp.max(best, axis=1, keepdims=True)
    idx_m = jnp.min(jnp.where(best == maxv, bestcol, _IMAX),
                    axis=1, keepdims=True)
    yt = xt + _gumbel_bits(rowbase + _TAIL0 + colt)
    maxv_t = jnp.max(yt, axis=1, keepdims=True)
    idx_t = jnp.min(jnp.where(yt == maxv_t, colt + _TAIL0, _IMAX),
                    axis=1, keepdims=True)
    tw = maxv_t > maxv
    idx = jnp.where(tw, idx_t, idx_m)
    idx_ref[...] = idx

    def write_body(c, ap):
        xs = x_ref[:, pl.ds(c * _CH, _CH)]
        col = colc + c * _CH
        lp = xs - lse
        lp_ref[:, pl.ds(c * _CH, _CH)] = lp
        return jnp.maximum(ap, jnp.where(col == idx, lp, _NEG_INF))

    ap = jax.lax.fori_loop(
        0, _NCH, write_body, jnp.full((_R, _CH), _NEG_INF, jnp.float32))
    alp = jnp.max(ap, axis=1, keepdims=True)
    lpt = xt - lse
    lp_ref[:, pl.ds(_TAIL0, _TAIL)] = lpt
    alp_t = jnp.max(jnp.where(colt + _TAIL0 == idx, lpt, _NEG_INF),
                    axis=1, keepdims=True)
    alp_ref[...] = jnp.maximum(alp, alp_t)


def _sample(logits, interpret=False):
    return pl.pallas_call(
        _sample_body,
        grid=(_BT // _R,),
        in_specs=[pl.BlockSpec((_R, _A), lambda b: (b, 0))],
        out_specs=[
            pl.BlockSpec((_R, _A), lambda b: (b, 0)),
            pl.BlockSpec((_R, 1), lambda b: (b, 0)),
            pl.BlockSpec((_R, 1), lambda b: (b, 0)),
        ],
        out_shape=[
            jax.ShapeDtypeStruct((_BT, _A), jnp.float32),
            jax.ShapeDtypeStruct((_BT, 1), jnp.int32),
            jax.ShapeDtypeStruct((_BT, 1), jnp.float32),
        ],
        compiler_params=pltpu.CompilerParams(
            dimension_semantics=("parallel",),
        ),
        interpret=interpret,
    )(logits)


def kernel(logits, value, action_index_tensor):
    lp, idx, alp = _sample(logits)
    idx = idx.reshape(-1)
    actions = jnp.take(action_index_tensor, idx, axis=0)
    return actions, alp.reshape(-1), value.reshape(-1), lp


# speedup vs baseline: 1.5354x; 1.1151x over previous
import functools

import jax
import jax.numpy as jnp
from jax import lax
from jax.experimental import pallas as pl
from jax.experimental.pallas import tpu as pltpu
from jax.experimental.pallas import tpu_sc as plsc

_BT = 256
_A = 100000
_R = 8
_CH = 512
_U = 4
_NCH = _A // _CH
_TAIL = _A - _NCH * _CH
_TAIL0 = _NCH * _CH

_KS0 = 0
_KS1 = 42
_KS2 = 0 ^ 42 ^ 0x1BD11BDA
_KS = (_KS0, _KS1, _KS2)
_ROTS = ((13, 15, 26, 6), (17, 29, 16, 24))
_TINY = float(jnp.finfo(jnp.float32).tiny)
_NEG_INF = float("-inf")
_IMAX = jnp.iinfo(jnp.int32).max


def _rotl(x, r):
    return lax.shift_left(x, r) | lax.shift_right_logical(x, 32 - r)


def _gumbel_bits(i):
    x1 = i + _KS1
    x0 = x1
    first = True
    for r in range(5):
        for rot in _ROTS[r % 2]:
            if first:
                first = False
            else:
                x0 = x0 + x1
            x1 = _rotl(x1, rot)
            x1 = x1 ^ x0
        x0 = x0 + _KS[(r + 1) % 3]
        x1 = x1 + (_KS[(r + 2) % 3] + r + 1)
    bits = x0 ^ x1
    fbits = lax.shift_right_logical(bits, 9) | 0x3F800000
    f = lax.bitcast_convert_type(fbits, jnp.float32) - 1.0
    u = jnp.maximum(_TINY, f * (1.0 - _TINY) + _TINY)
    return -jnp.log(-jnp.log(u))


def _sample_body(x_ref, lp_ref, idx_ref, alp_ref):
    b = pl.program_id(0)
    rowbase = (lax.broadcasted_iota(jnp.int32, (_R, 1), 0) + b * _R) * _A
    colc = lax.broadcasted_iota(jnp.int32, (_R, _CH), 1)
    colt = lax.broadcasted_iota(jnp.int32, (_R, _TAIL), 1)

    mp = jnp.full((_R, _CH), _NEG_INF, jnp.float32)
    for c in range(_NCH):
        mp = jnp.maximum(mp, x_ref[:, pl.ds(c * _CH, _CH)])
    m = jnp.max(mp, axis=1, keepdims=True)
    xt = x_ref[:, pl.ds(_TAIL0, _TAIL)]
    m = jnp.maximum(m, jnp.max(xt, axis=1, keepdims=True))

    sp = jnp.zeros((_R, _CH), jnp.float32)
    best = jnp.full((_R, _CH), _NEG_INF, jnp.float32)
    bestcol = jnp.zeros((_R, _CH), jnp.int32)
    for c in range(_NCH):
        xs = x_ref[:, pl.ds(c * _CH, _CH)]
        col = colc + c * _CH
        y = xs + _gumbel_bits(rowbase + col)
        upd = y > best
        best = jnp.where(upd, y, best)
        bestcol = jnp.where(upd, col, bestcol)
        sp = sp + jnp.exp(xs - m)

    s = jnp.sum(sp, axis=1, keepdims=True)
    s = s + jnp.sum(jnp.exp(xt - m), axis=1, keepdims=True)
    lse = m + jnp.log(s)

    maxv = jnp.max(best, axis=1, keepdims=True)
    idx_m = jnp.min(jnp.where(best == maxv, bestcol, _IMAX),
                    axis=1, keepdims=True)
    yt = xt + _gumbel_bits(rowbase + _TAIL0 + colt)
    maxv_t = jnp.max(yt, axis=1, keepdims=True)
    idx_t = jnp.min(jnp.where(yt == maxv_t, colt + _TAIL0, _IMAX),
                    axis=1, keepdims=True)
    tw = maxv_t > maxv
    idx = jnp.where(tw, idx_t, idx_m)
    idx_ref[...] = idx

    ap = jnp.full((_R, _CH), _NEG_INF, jnp.float32)
    for c in range(_NCH):
        xs = x_ref[:, pl.ds(c * _CH, _CH)]
        col = colc + c * _CH
        lp = xs - lse
        lp_ref[:, pl.ds(c * _CH, _CH)] = lp
        ap = jnp.maximum(ap, jnp.where(col == idx, lp, _NEG_INF))
    alp = jnp.max(ap, axis=1, keepdims=True)
    lpt = xt - lse
    lp_ref[:, pl.ds(_TAIL0, _TAIL)] = lpt
    alp_t = jnp.max(jnp.where(colt + _TAIL0 == idx, lpt, _NEG_INF),
                    axis=1, keepdims=True)
    alp_ref[...] = jnp.maximum(alp, alp_t)


def _sample(logits, interpret=False):
    return pl.pallas_call(
        _sample_body,
        grid=(_BT // _R,),
        in_specs=[pl.BlockSpec((_R, _A), lambda b: (b, 0))],
        out_specs=[
            pl.BlockSpec((_R, _A), lambda b: (b, 0)),
            pl.BlockSpec((_R, 1), lambda b: (b, 0)),
            pl.BlockSpec((_R, 1), lambda b: (b, 0)),
        ],
        out_shape=[
            jax.ShapeDtypeStruct((_BT, _A), jnp.float32),
            jax.ShapeDtypeStruct((_BT, 1), jnp.int32),
            jax.ShapeDtypeStruct((_BT, 1), jnp.float32),
        ],
        compiler_params=pltpu.CompilerParams(
            dimension_semantics=("parallel",),
        ),
        interpret=interpret,
    )(logits)


def kernel(logits, value, action_index_tensor):
    lp, idx, alp = _sample(logits)
    idx = idx.reshape(-1)
    actions = jnp.take(action_index_tensor, idx, axis=0)
    return actions, alp.reshape(-1), value.reshape(-1), lp
